# baseline (speedup 1.0000x reference)
import numpy as np
import jax
import jax.numpy as jnp
from jax.experimental import pallas as pl

_SIGMA = 0.2
_BATCH = 64
_H, _W, _C = 56, 56, 192
_HW = _H * _W
_BH = 112
_NCHUNK = _HW // _BH

_PERM = [17, 27, 42, 32, 1, 3, 58, 51, 40, 28, 52, 19, 9, 33, 11, 45,
         31, 5, 15, 39, 50, 47, 20, 0, 46, 14, 49, 44, 38, 61, 2, 54,
         36, 35, 62, 63, 21, 59, 30, 43, 22, 18, 24, 26, 53, 12, 16, 6,
         7, 57, 55, 48, 13, 37, 60, 10, 29, 34, 25, 56, 4, 41, 23, 8]


def _mix_body(x_ref, r_ref, t_ref, o_ref):
    r = r_ref[...]
    t = t_ref[...]
    a = 1.0 + r * jnp.cos(t)
    b = r * jnp.sin(t)
    for i in range(_BATCH):
        o_ref[i] = x_ref[i] * a + x_ref[_PERM[i]] * b


def _draws():
    key = jax.random.key(42)
    _, k_r, k_theta = jax.random.split(key, 3)
    rs = (_HW * _C // 128, 128)
    r = jax.random.normal(k_r, rs, dtype=jnp.float16) * jnp.float16(_SIGMA)
    theta = jax.random.uniform(
        k_theta, rs, dtype=jnp.float16, minval=-np.pi, maxval=np.pi)
    return (r.astype(jnp.float32).reshape(_HW, _C),
            theta.astype(jnp.float32).reshape(_HW, _C))


def kernel(inputs):
    x = inputs.reshape(_BATCH, _HW, _C)
    r, theta = _draws()
    y = pl.pallas_call(
        _mix_body,
        grid=(_NCHUNK,),
        in_specs=[
            pl.BlockSpec((_BATCH, _BH, _C), lambda c: (0, c, 0)),
            pl.BlockSpec((_BH, _C), lambda c: (c, 0)),
            pl.BlockSpec((_BH, _C), lambda c: (c, 0)),
        ],
        out_specs=pl.BlockSpec((_BATCH, _BH, _C), lambda c: (0, c, 0)),
        out_shape=jax.ShapeDtypeStruct((_BATCH, _HW, _C), jnp.float32),
    )(x, r, theta)
    return y.reshape(inputs.shape)
